# baseline (speedup 1.0000x reference)
import functools

import jax
import jax.numpy as jnp
from jax import lax
from jax.experimental import pallas as pl
from jax.experimental.pallas import tpu as pltpu
from jax.experimental.pallas import tpu_sc as plsc

TMR = 256


def _routing_metadata(expert_weights, top_k_indices, T, K, E):
    P = T * K
    e_flat = top_k_indices.reshape(P).astype(jnp.int32)
    w_flat = expert_weights.reshape(P)
    onehot = (e_flat[:, None] == jnp.arange(E, dtype=jnp.int32)[None, :]).astype(jnp.int32)
    csum = jnp.cumsum(onehot, axis=0)
    counts = csum[-1]
    rank = jnp.take_along_axis(csum, e_flat[:, None], axis=1)[:, 0] - 1
    padded_counts = ((counts + TMR - 1) // TMR) * TMR
    cum_padded = jnp.cumsum(padded_counts)
    padded_offsets = cum_padded - padded_counts
    pos = padded_offsets[e_flat] + rank
    J = P + E * TMR
    NT = J // TMR
    row_token = jnp.zeros((J,), jnp.int32).at[pos].set(
        jnp.arange(P, dtype=jnp.int32) // K)
    row_w = jnp.zeros((J,), jnp.float32).at[pos].set(w_flat)
    tile_starts = jnp.arange(NT, dtype=jnp.int32) * TMR
    tile_expert = jnp.minimum(
        jnp.searchsorted(cum_padded, tile_starts, side="right").astype(jnp.int32),
        E - 1)
    pos2 = pos.reshape(T, K)
    return row_token, row_w, tile_expert, pos2[:, 0], pos2[:, 1], J, NT


def _sc_gather(x2, row_token, J, D):
    info = plsc.get_sparse_core_info()
    NC, NS = info.num_cores, info.num_subcores
    NW = NC * NS
    rows_per_w = J // NW
    CH = 16
    nchunk = rows_per_w // CH
    mesh = plsc.VectorSubcoreMesh(core_axis_name="c", subcore_axis_name="s")

    @functools.partial(
        pl.kernel, mesh=mesh,
        out_type=jax.ShapeDtypeStruct((J, D), jnp.float32),
        scratch_types=[
            pltpu.VMEM((CH,), jnp.int32),
            pltpu.VMEM((CH, D), jnp.float32),
            pltpu.SemaphoreType.DMA,
        ],
    )
    def gather_k(x_hbm, rt_hbm, out_hbm, idx_v, rows_v, sem):
        wid = lax.axis_index("s") * NC + lax.axis_index("c")
        base0 = wid * rows_per_w

        def body(c, _):
            base = base0 + c * CH
            pltpu.sync_copy(rt_hbm.at[pl.ds(base, CH)], idx_v)
            pltpu.async_copy(x_hbm.at[idx_v], rows_v, sem).wait()
            pltpu.sync_copy(rows_v, out_hbm.at[pl.ds(base, CH)])
            return 0

        lax.fori_loop(0, nchunk, body, 0)

    return gather_k(x2, row_token)


def _tc_matmul_body(te_ref, xg_ref, rw_ref, w_ref, b_ref, o_ref):
    xb = xg_ref[...].astype(jnp.bfloat16)
    mm = lax.dot_general(xb, w_ref[0], (((1,), (1,)), ((), ())),
                         preferred_element_type=jnp.float32)
    o_ref[...] = rw_ref[...] * (mm + b_ref[0])


def _tc_ragged_matmul(xg, row_w, tile_expert, Wb, b, J, NT, D, O, E):
    grid_spec = pltpu.PrefetchScalarGridSpec(
        num_scalar_prefetch=1,
        grid=(NT,),
        in_specs=[
            pl.BlockSpec((TMR, D), lambda i, te: (i, 0)),
            pl.BlockSpec((TMR, 1), lambda i, te: (i, 0)),
            pl.BlockSpec((1, O, D), lambda i, te: (te[i], 0, 0)),
            pl.BlockSpec((1, 1, O), lambda i, te: (te[i], 0, 0)),
        ],
        out_specs=pl.BlockSpec((TMR, O), lambda i, te: (i, 0)),
    )
    return pl.pallas_call(
        _tc_matmul_body,
        grid_spec=grid_spec,
        out_shape=jax.ShapeDtypeStruct((J, O), jnp.float32),
    )(tile_expert, xg, row_w.reshape(J, 1), Wb, b.reshape(E, 1, O))


def _sc_combine(yg, pos0, pos1, T, O):
    info = plsc.get_sparse_core_info()
    NC, NS = info.num_cores, info.num_subcores
    NW = NC * NS
    tok_per_w = T // NW
    CH = 16
    nchunk = tok_per_w // CH
    nvec = O // 16
    mesh = plsc.VectorSubcoreMesh(core_axis_name="c", subcore_axis_name="s")

    @functools.partial(
        pl.kernel, mesh=mesh,
        out_type=jax.ShapeDtypeStruct((T, O), jnp.float32),
        scratch_types=[
            pltpu.VMEM((CH,), jnp.int32),
            pltpu.VMEM((CH,), jnp.int32),
            pltpu.VMEM((CH, O), jnp.float32),
            pltpu.VMEM((CH, O), jnp.float32),
            pltpu.SemaphoreType.DMA,
            pltpu.SemaphoreType.DMA,
        ],
    )
    def combine_k(yg_hbm, p0_hbm, p1_hbm, out_hbm, i0_v, i1_v, r0_v, r1_v,
                  sem0, sem1):
        wid = lax.axis_index("s") * NC + lax.axis_index("c")
        base0 = wid * tok_per_w

        def body(c, _):
            base = base0 + c * CH
            pltpu.sync_copy(p0_hbm.at[pl.ds(base, CH)], i0_v)
            pltpu.sync_copy(p1_hbm.at[pl.ds(base, CH)], i1_v)
            cp0 = pltpu.async_copy(yg_hbm.at[i0_v], r0_v, sem0)
            cp1 = pltpu.async_copy(yg_hbm.at[i1_v], r1_v, sem1)
            cp0.wait()
            cp1.wait()

            def vadd(i, _):
                r = i // nvec
                col = (i % nvec) * 16
                col = pl.multiple_of(col, 16)
                r0_v[r, pl.ds(col, 16)] = (r0_v[r, pl.ds(col, 16)]
                                           + r1_v[r, pl.ds(col, 16)])
                return 0

            lax.fori_loop(0, CH * nvec, vadd, 0)
            pltpu.sync_copy(r0_v, out_hbm.at[pl.ds(base, CH)])
            return 0

        lax.fori_loop(0, nchunk, body, 0)

    return combine_k(yg, pos0, pos1)


def kernel(x, expert_weights, top_k_indices, W, b):
    B, S, D = x.shape
    E, O, _ = W.shape
    K = expert_weights.shape[-1]
    T = B * S

    x2 = x.reshape(T, D)
    Wb = W.astype(jnp.bfloat16)

    row_token, row_w, tile_expert, pos0, pos1, J, NT = _routing_metadata(
        expert_weights, top_k_indices, T, K, E)

    xg = _sc_gather(x2, row_token, J, D)
    yg = _tc_ragged_matmul(xg, row_w, tile_expert, Wb, b, J, NT, D, O, E)
    out = _sc_combine(yg, pos0, pos1, T, O)
    return out.reshape(B, S, O)


# speedup vs baseline: 1.0554x; 1.0554x over previous
import functools

import jax
import jax.numpy as jnp
from jax import lax
from jax.experimental import pallas as pl
from jax.experimental.pallas import tpu as pltpu
from jax.experimental.pallas import tpu_sc as plsc

TMR = 256


def _routing_metadata(expert_weights, top_k_indices, T, K, E):
    P = T * K
    e_flat = top_k_indices.reshape(P).astype(jnp.int32)
    w_flat = expert_weights.reshape(P)
    onehot = (e_flat[:, None] == jnp.arange(E, dtype=jnp.int32)[None, :]).astype(jnp.int32)
    csum = jnp.cumsum(onehot, axis=0)
    counts = csum[-1]
    rank = jnp.take_along_axis(csum, e_flat[:, None], axis=1)[:, 0] - 1
    padded_counts = ((counts + TMR - 1) // TMR) * TMR
    cum_padded = jnp.cumsum(padded_counts)
    padded_offsets = cum_padded - padded_counts
    pos = padded_offsets[e_flat] + rank
    J = P + E * TMR
    NT = J // TMR
    row_token = jnp.zeros((J,), jnp.int32).at[pos].set(
        jnp.arange(P, dtype=jnp.int32) // K)
    row_w = jnp.zeros((J,), jnp.float32).at[pos].set(w_flat)
    tile_starts = jnp.arange(NT, dtype=jnp.int32) * TMR
    tile_expert = jnp.minimum(
        jnp.searchsorted(cum_padded, tile_starts, side="right").astype(jnp.int32),
        E - 1)
    pos2 = pos.reshape(T, K)
    return row_token, row_w, tile_expert, pos2[:, 0], pos2[:, 1], J, NT


def _sc_gather(x2, row_token, J, D):
    info = plsc.get_sparse_core_info()
    NC, NS = info.num_cores, info.num_subcores
    NW = NC * NS
    rows_per_w = J // NW
    CH = 24
    nchunk = rows_per_w // CH
    mesh = plsc.VectorSubcoreMesh(core_axis_name="c", subcore_axis_name="s")

    @functools.partial(
        pl.kernel, mesh=mesh,
        out_type=jax.ShapeDtypeStruct((J, D), jnp.float32),
        scratch_types=[
            pltpu.VMEM((2, CH), jnp.int32),
            pltpu.VMEM((2, CH, D), jnp.float32),
            pltpu.SemaphoreType.DMA,
            pltpu.SemaphoreType.DMA,
            pltpu.SemaphoreType.DMA,
            pltpu.SemaphoreType.DMA,
        ],
    )
    def gather_k(x_hbm, rt_hbm, out_hbm, idx_v, rows_v, g0, g1, s0, s1):
        wid = lax.axis_index("s") * NC + lax.axis_index("c")
        base0 = wid * rows_per_w
        gsem = (g0, g1)
        ssem = (s0, s1)

        def start(c, buf):
            base = base0 + c * CH
            pltpu.sync_copy(rt_hbm.at[pl.ds(base, CH)], idx_v.at[buf])
            pltpu.async_copy(x_hbm.at[idx_v.at[buf]], rows_v.at[buf], gsem[buf])

        def wait_scatter(buf):
            pltpu.make_async_copy(
                rows_v.at[buf], out_hbm.at[pl.ds(base0, CH)], ssem[buf]).wait()

        start(0, 0)

        def body(c, _):
            cur = lax.rem(c, 2)

            @pl.when(c + 1 < nchunk)
            def _():
                @pl.when(c >= 1)
                def _():
                    pl.when(cur == 0)(lambda: wait_scatter(1))
                    pl.when(cur == 1)(lambda: wait_scatter(0))
                pl.when(cur == 0)(lambda: start(c + 1, 1))
                pl.when(cur == 1)(lambda: start(c + 1, 0))

            def finish(buf):
                pltpu.make_async_copy(
                    x_hbm.at[idx_v.at[buf]], rows_v.at[buf], gsem[buf]).wait()
                base = base0 + c * CH
                pltpu.async_copy(rows_v.at[buf], out_hbm.at[pl.ds(base, CH)],
                                 ssem[buf])

            pl.when(cur == 0)(lambda: finish(0))
            pl.when(cur == 1)(lambda: finish(1))
            return 0

        lax.fori_loop(0, nchunk, body, 0)
        wait_scatter((nchunk - 2) % 2)
        wait_scatter((nchunk - 1) % 2)

    return gather_k(x2, row_token)


def _tc_matmul_body(te_ref, xg_ref, rw_ref, w_ref, b_ref, o_ref):
    xb = xg_ref[...].astype(jnp.bfloat16)
    mm = lax.dot_general(xb, w_ref[0], (((1,), (1,)), ((), ())),
                         preferred_element_type=jnp.float32)
    o_ref[...] = rw_ref[...] * (mm + b_ref[0])


def _tc_ragged_matmul(xg, row_w, tile_expert, Wb, b, J, NT, D, O, E):
    grid_spec = pltpu.PrefetchScalarGridSpec(
        num_scalar_prefetch=1,
        grid=(NT,),
        in_specs=[
            pl.BlockSpec((TMR, D), lambda i, te: (i, 0)),
            pl.BlockSpec((TMR, 1), lambda i, te: (i, 0)),
            pl.BlockSpec((1, O, D), lambda i, te: (te[i], 0, 0)),
            pl.BlockSpec((1, 1, O), lambda i, te: (te[i], 0, 0)),
        ],
        out_specs=pl.BlockSpec((TMR, O), lambda i, te: (i, 0)),
    )
    return pl.pallas_call(
        _tc_matmul_body,
        grid_spec=grid_spec,
        out_shape=jax.ShapeDtypeStruct((J, O), jnp.float32),
    )(tile_expert, xg, row_w.reshape(J, 1), Wb, b.reshape(E, 1, O))


def _sc_combine(yg, pos0, pos1, T, O):
    info = plsc.get_sparse_core_info()
    NC, NS = info.num_cores, info.num_subcores
    NW = NC * NS
    tok_per_w = T // NW
    CH = 8
    nchunk = tok_per_w // CH
    nvec = O // 16
    mesh = plsc.VectorSubcoreMesh(core_axis_name="c", subcore_axis_name="s")

    @functools.partial(
        pl.kernel, mesh=mesh,
        out_type=jax.ShapeDtypeStruct((T, O), jnp.float32),
        scratch_types=[
            pltpu.VMEM((2, CH), jnp.int32),
            pltpu.VMEM((2, CH), jnp.int32),
            pltpu.VMEM((2, CH, O), jnp.float32),
            pltpu.VMEM((2, CH, O), jnp.float32),
            pltpu.SemaphoreType.DMA,
            pltpu.SemaphoreType.DMA,
            pltpu.SemaphoreType.DMA,
            pltpu.SemaphoreType.DMA,
            pltpu.SemaphoreType.DMA,
            pltpu.SemaphoreType.DMA,
        ],
    )
    def combine_k(yg_hbm, p0_hbm, p1_hbm, out_hbm, i0_v, i1_v, r0_v, r1_v,
                  ga0, ga1, gb0, gb1, s0, s1):
        wid = lax.axis_index("s") * NC + lax.axis_index("c")
        base0 = wid * tok_per_w
        gasem = (ga0, ga1)
        gbsem = (gb0, gb1)
        ssem = (s0, s1)

        def start(c, buf):
            base = base0 + c * CH
            pltpu.sync_copy(p0_hbm.at[pl.ds(base, CH)], i0_v.at[buf])
            pltpu.sync_copy(p1_hbm.at[pl.ds(base, CH)], i1_v.at[buf])
            pltpu.async_copy(yg_hbm.at[i0_v.at[buf]], r0_v.at[buf], gasem[buf])
            pltpu.async_copy(yg_hbm.at[i1_v.at[buf]], r1_v.at[buf], gbsem[buf])

        def wait_scatter(buf):
            pltpu.make_async_copy(
                r0_v.at[buf], out_hbm.at[pl.ds(base0, CH)], ssem[buf]).wait()

        start(0, 0)

        def body(c, _):
            cur = lax.rem(c, 2)

            @pl.when(c + 1 < nchunk)
            def _():
                @pl.when(c >= 1)
                def _():
                    pl.when(cur == 0)(lambda: wait_scatter(1))
                    pl.when(cur == 1)(lambda: wait_scatter(0))
                pl.when(cur == 0)(lambda: start(c + 1, 1))
                pl.when(cur == 1)(lambda: start(c + 1, 0))

            def finish(buf):
                pltpu.make_async_copy(
                    yg_hbm.at[i0_v.at[buf]], r0_v.at[buf], gasem[buf]).wait()
                pltpu.make_async_copy(
                    yg_hbm.at[i1_v.at[buf]], r1_v.at[buf], gbsem[buf]).wait()

                def vadd(i, _):
                    r = i // nvec
                    col = (i % nvec) * 16
                    col = pl.multiple_of(col, 16)
                    r0_v[buf, r, pl.ds(col, 16)] = (
                        r0_v[buf, r, pl.ds(col, 16)]
                        + r1_v[buf, r, pl.ds(col, 16)])
                    return 0

                lax.fori_loop(0, CH * nvec, vadd, 0)
                base = base0 + c * CH
                pltpu.async_copy(r0_v.at[buf], out_hbm.at[pl.ds(base, CH)],
                                 ssem[buf])

            pl.when(cur == 0)(lambda: finish(0))
            pl.when(cur == 1)(lambda: finish(1))
            return 0

        lax.fori_loop(0, nchunk, body, 0)
        wait_scatter((nchunk - 2) % 2)
        wait_scatter((nchunk - 1) % 2)

    return combine_k(yg, pos0, pos1)


def kernel(x, expert_weights, top_k_indices, W, b):
    B, S, D = x.shape
    E, O, _ = W.shape
    K = expert_weights.shape[-1]
    T = B * S

    x2 = x.reshape(T, D)
    Wb = W.astype(jnp.bfloat16)

    row_token, row_w, tile_expert, pos0, pos1, J, NT = _routing_metadata(
        expert_weights, top_k_indices, T, K, E)

    xg = _sc_gather(x2, row_token, J, D)
    yg = _tc_ragged_matmul(xg, row_w, tile_expert, Wb, b, J, NT, D, O, E)
    out = _sc_combine(yg, pos0, pos1, T, O)
    return out.reshape(B, S, O)
